# baseline (speedup 1.0000x reference)
import dataclasses
import functools

import jax
import jax.numpy as jnp
import numpy as np
from jax import lax
from jax.experimental import pallas as pl
from jax.experimental.pallas import tpu as pltpu
from jax.experimental.pallas import tpu_sc as plsc

N_NODES = 10000
N_EDGES = 320000
NODE_DIM = 128
EDGE_DIM = 16
HIDDEN = 128

NCORES = 2
NSUB = 16
NTILES = NCORES * NSUB
CB = 56
NCHUNKS = 184
NT = NCHUNKS // 4
T_EDGES = CB * NCHUNKS
E_PAD = NTILES * T_EDGES
PK = NODE_DIM // 2
N_ACC = 10240
ROWS_PER_TILE = N_ACC // NSUB
N_TAB = 10016
CW = 16
CCB = 80
CNCHUNKS = (N_EDGES // NTILES) // CCB

_PERM = np.empty(HIDDEN, np.int32)
for _g in range(HIDDEN // 32):
    for _i in range(16):
        _PERM[_g * 32 + _i] = _g * 32 + 2 * _i
        _PERM[_g * 32 + 16 + _i] = _g * 32 + 2 * _i + 1


def _node_mlp_kernel(x_ref, w1a_ref, w1b_ref, b1_ref, xa_ref, xb_ref):
    x = x_ref[...]
    xa = jnp.dot(x, w1a_ref[...], preferred_element_type=jnp.float32) + b1_ref[...]
    xb = jnp.dot(x, w1b_ref[...], preferred_element_type=jnp.float32)
    xa_ref[...] = xa.astype(jnp.bfloat16)
    xb_ref[...] = xb.astype(jnp.bfloat16)


def _edge_mlp_kernel(ea_ref, w1c_ref, out_ref):
    out_ref[...] = jnp.dot(
        ea_ref[...], w1c_ref[...], preferred_element_type=jnp.float32
    ).astype(jnp.bfloat16)


def _sc_count_kernel(dst_hbm, cnt_hbm, idxd, ones_b, cnt_sh, sem):
    c = lax.axis_index("c")
    s = lax.axis_index("s")
    tile = c * NSUB + s
    ebase = tile * (N_EDGES // NTILES)
    rbase = s * 640

    @pl.loop(0, CCB)
    def _(r):
        ones_b.at[pl.ds(r, 1), pl.ds(0, CW)][...] = jnp.zeros((1, CW), jnp.float32)
    for k in range(8):
        @pl.when(rbase + k * CCB < N_NODES)
        def _(k=k):
            pltpu.sync_copy(ones_b, cnt_sh.at[pl.ds(rbase + k * CCB, CCB), :])

    @pl.loop(0, CCB)
    def _(r):
        ones_b.at[pl.ds(r, 1), pl.ds(0, CW)][...] = jnp.full((1, CW), 1.0, jnp.float32)
    plsc.subcore_barrier()

    @pl.loop(0, CNCHUNKS)
    def _(j):
        pltpu.sync_copy(dst_hbm.at[pl.ds(ebase + j * CCB, CCB)], idxd)
        pltpu.sync_copy(ones_b, cnt_sh.at[idxd], add=True)

    plsc.subcore_barrier()
    for k in range(8):
        @pl.when(rbase + k * CCB < N_NODES)
        def _(k=k):
            pltpu.sync_copy(cnt_sh.at[pl.ds(rbase + k * CCB, CCB), :],
                            cnt_hbm.at[pl.ds(c * N_NODES + rbase + k * CCB, CCB), :])


def _sc_edge_kernel(dst_hbm, src_hbm, ea_hbm, xa_hbm, xb_hbm,
                    hacc_hbm,
                    idxd0, idxd1, idxd2, idxd3,
                    idxs0, idxs1, idxs2, idxs3,
                    bufa0, bufa1, bufb0, bufb1, bufe0, bufe1, bufh0, bufh1,
                    acc_sh,
                    sem_i0, sem_i1, sem_i2, sem_i3,
                    sem_e0, sem_e1, sem_a0, sem_a1,
                    sem_b0, sem_b1, sem_s0, sem_s1):
    cx = lax.axis_index("c")
    sx = lax.axis_index("s")
    tile = cx * NSUB + sx
    ebase = tile * T_EDGES

    idxd = (idxd0, idxd1, idxd2, idxd3)
    idxs = (idxs0, idxs1, idxs2, idxs3)
    bufa = (bufa0, bufa1)
    bufb = (bufb0, bufb1)
    bufe = (bufe0, bufe1)
    bufh = (bufh0, bufh1)
    sem_i = (sem_i0, sem_i1, sem_i2, sem_i3)
    sem_e = (sem_e0, sem_e1)
    sem_a = (sem_a0, sem_a1)
    sem_b = (sem_b0, sem_b1)
    sem_s = (sem_s0, sem_s1)

    def i_descs(c, q):
        return (
            pltpu.make_async_copy(dst_hbm.at[pl.ds(ebase + c * CB, CB)], idxd[q], sem_i[q]),
            pltpu.make_async_copy(src_hbm.at[pl.ds(ebase + c * CB, CB)], idxs[q], sem_i[q]),
        )

    def e_desc(c, p):
        return pltpu.make_async_copy(
            ea_hbm.at[pl.ds(ebase + c * CB, CB), :], bufe[p], sem_e[p])

    def a_desc(q, p):
        return pltpu.make_async_copy(xa_hbm.at[idxd[q]], bufa[p], sem_a[p])

    def b_desc(q, p):
        return pltpu.make_async_copy(xb_hbm.at[idxs[q]], bufb[p], sem_b[p])

    def s_desc(q, p):
        return pltpu.make_async_copy(bufh[p], acc_sh.at[idxd[q]], sem_s[p])

    @pl.loop(0, CB)
    def _(r):
        for cc in range(NODE_DIM // 16):
            bufh0.at[r, pl.ds(cc * 16, 16)][...] = jnp.zeros((16,), jnp.float32)
    rbase = sx * ROWS_PER_TILE
    pltpu.sync_copy(bufh0, acc_sh.at[pl.ds(rbase, CB), :])
    for k in range(1, (ROWS_PER_TILE + CB - 1) // CB):
        rows = min(CB, ROWS_PER_TILE - k * CB)
        pltpu.sync_copy(bufh0.at[pl.ds(0, rows), :],
                        acc_sh.at[pl.ds(rbase + k * CB, rows), :])
    plsc.subcore_barrier()

    def compute(p):
        @pl.loop(0, CB)
        def _(r):
            for g in range(PK // 16):
                wa = bufa[p].at[r, pl.ds(g * 16, 16)][...]
                wb = bufb[p].at[r, pl.ds(g * 16, 16)][...]
                we = bufe[p].at[r, pl.ds(g * 16, 16)][...]
                zero = jnp.zeros((16,), jnp.float32)
                mask = jnp.full((16,), jnp.int32(-65536))
                lo = (plsc.bitcast(wa << 16, jnp.float32)
                      + plsc.bitcast(wb << 16, jnp.float32)
                      + plsc.bitcast(we << 16, jnp.float32))
                hi = (plsc.bitcast(wa & mask, jnp.float32)
                      + plsc.bitcast(wb & mask, jnp.float32)
                      + plsc.bitcast(we & mask, jnp.float32))
                bufh[p].at[r, pl.ds(g * 32, 16)][...] = jnp.maximum(lo, zero)
                bufh[p].at[r, pl.ds(g * 32 + 16, 16)][...] = jnp.maximum(hi, zero)

    for d in i_descs(0, 0):
        d.start()
    for d in i_descs(1, 1):
        d.start()
    for d in i_descs(2, 2):
        d.start()
    for d in i_descs(0, 0):
        d.wait()
    a_desc(0, 0).start()
    b_desc(0, 0).start()
    e_desc(0, 0).start()

    @pl.loop(0, NT)
    def _(t):
        for u in range(4):
            c = 4 * t + u
            p = u % 2
            p1 = (u + 1) % 2
            q1 = (u + 1) % 4

            def s1(c=c, p1=p1, q1=q1):
                for d in i_descs(c + 1, q1):
                    d.wait()
                a_desc(q1, p1).start()
                b_desc(q1, p1).start()
                e_desc(c + 1, p1).start()

            if u == 3:
                pl.when(t < NT - 1)(s1)
            else:
                s1()

            a_desc(u, p).wait()
            b_desc(u, p).wait()
            e_desc(c, p).wait()

            def s3w(p1=p1, u=u):
                s_desc((u + 3) % 4, p1).wait()

            if u == 0:
                pl.when(t > 0)(s3w)
            else:
                s3w()

            def s3i(c=c, u=u):
                for d in i_descs(c + 3, (u + 3) % 4):
                    d.start()

            if u == 0:
                s3i()
            else:
                pl.when(t < NT - 1)(s3i)

            compute(p)
            s_desc(u, p).start(add=True)

    s_desc(3, 1).wait()
    plsc.subcore_barrier()
    pltpu.sync_copy(acc_sh.at[pl.ds(rbase, ROWS_PER_TILE), :],
                    hacc_hbm.at[pl.ds(cx * N_ACC + rbase, ROWS_PER_TILE), :])


def _combine_kernel(hacc_ref, cnt_ref, w2_ref, b2_ref, out_ref):
    h = hacc_ref[:N_NODES, :] + hacc_ref[N_ACC:N_ACC + N_NODES, :]
    cnt = cnt_ref[:N_NODES, 0:1] + cnt_ref[N_NODES:, 0:1]
    out_ref[...] = (
        jnp.dot(h, w2_ref[...], preferred_element_type=jnp.float32) + cnt * b2_ref[...]
    )


def kernel(x, edge_index, edge_attr, W1, b1, W2, b2):
    dst = edge_index[1].astype(jnp.int32)
    src = edge_index[0].astype(jnp.int32)
    W1a = W1[:NODE_DIM]
    W1b = W1[NODE_DIM:2 * NODE_DIM]
    W1c = W1[2 * NODE_DIM:]
    b1r = b1.reshape(1, HIDDEN)
    b2r = b2.reshape(1, NODE_DIM)

    per_tile = N_EDGES // NTILES
    pad_n = T_EDGES - per_tile
    dst_f = jnp.pad(dst.reshape(NTILES, per_tile), ((0, 0), (0, pad_n)),
                    constant_values=N_NODES).reshape(E_PAD)
    src_f = jnp.pad(src.reshape(NTILES, per_tile), ((0, 0), (0, pad_n)),
                    constant_values=N_NODES).reshape(E_PAD)
    ea_in = jnp.pad(edge_attr.reshape(NTILES, per_tile, EDGE_DIM),
                    ((0, 0), (0, pad_n), (0, 0))).reshape(E_PAD, EDGE_DIM)

    xa, xb = pl.pallas_call(
        _node_mlp_kernel,
        out_shape=[
            jax.ShapeDtypeStruct((N_NODES, HIDDEN), jnp.bfloat16),
            jax.ShapeDtypeStruct((N_NODES, HIDDEN), jnp.bfloat16),
        ],
    )(x, W1a, W1b, b1r)
    xa = jnp.pad(xa, ((0, N_TAB - N_NODES), (0, 0)))
    xb = jnp.pad(xb, ((0, N_TAB - N_NODES), (0, 0)))
    xa_p = lax.bitcast_convert_type(xa.reshape(N_TAB, PK, 2), jnp.int32)
    xb_p = lax.bitcast_convert_type(xb.reshape(N_TAB, PK, 2), jnp.int32)

    eb = E_PAD // 16
    ea = pl.pallas_call(
        _edge_mlp_kernel,
        grid=(16,),
        in_specs=[
            pl.BlockSpec((eb, EDGE_DIM), lambda k: (k, 0)),
            pl.BlockSpec((EDGE_DIM, HIDDEN), lambda k: (0, 0)),
        ],
        out_specs=pl.BlockSpec((eb, HIDDEN), lambda k: (k, 0)),
        out_shape=jax.ShapeDtypeStruct((E_PAD, HIDDEN), jnp.bfloat16),
    )(ea_in, W1c)
    ea_p = lax.bitcast_convert_type(ea.reshape(E_PAD, PK, 2), jnp.int32)

    mesh = plsc.VectorSubcoreMesh(core_axis_name="c", subcore_axis_name="s")
    sc_count = pl.kernel(
        _sc_count_kernel,
        out_type=jax.ShapeDtypeStruct((NCORES * N_NODES, CW), jnp.float32),
        mesh=mesh,
        scratch_types=[
            pltpu.VMEM((CCB,), jnp.int32),
            pltpu.VMEM((CCB, CW), jnp.float32),
            pltpu.VMEM_SHARED((N_NODES, CW), jnp.float32),
            pltpu.SemaphoreType.DMA,
        ],
    )
    cnt = sc_count(dst)

    cp = pltpu.CompilerParams()
    if "needs_layout_passes" in pltpu.CompilerParams.__dataclass_fields__:
        cp = dataclasses.replace(cp, needs_layout_passes=False)
    if "use_tc_tiling_on_sc" in pltpu.CompilerParams.__dataclass_fields__:
        cp = dataclasses.replace(cp, use_tc_tiling_on_sc=False)
    sc_edge = pl.kernel(
        _sc_edge_kernel,
        out_type=jax.ShapeDtypeStruct((NCORES * N_ACC, HIDDEN), jnp.float32),
        mesh=mesh,
        compiler_params=cp,
        scratch_types=(
            [pltpu.VMEM((CB,), jnp.int32)] * 8
            + [pltpu.VMEM((CB, PK), jnp.int32)] * 6
            + [pltpu.VMEM((CB, HIDDEN), jnp.float32)] * 2
            + [pltpu.VMEM_SHARED((N_ACC, HIDDEN), jnp.float32)]
            + [pltpu.SemaphoreType.DMA] * 12
        ),
    )
    hacc = sc_edge(dst_f, src_f, ea_p, xa_p, xb_p)

    out = pl.pallas_call(
        _combine_kernel,
        out_shape=jax.ShapeDtypeStruct((N_NODES, NODE_DIM), jnp.float32),
    )(hacc, cnt, W2[_PERM], b2r)
    return out


# speedup vs baseline: 2.1845x; 2.1845x over previous
import functools

import jax
import jax.numpy as jnp
from jax import lax
from jax.experimental import pallas as pl
from jax.experimental.pallas import tpu as pltpu
from jax.experimental.pallas import tpu_sc as plsc

N_NODES = 10000
N_EDGES = 320000
NODE_DIM = 128
EDGE_DIM = 16
HIDDEN = 128

NCORES = 2
NSUB = 16
NTILES = NCORES * NSUB
CB = 64
NCHUNKS = 160
NT = NCHUNKS // 4
T_EDGES = CB * NCHUNKS
E_PAD = NTILES * T_EDGES
N_ACC = 10240
ROWS_PER_TILE = N_ACC // NSUB
N_TAB = 10016
CW = 16
CCB = 128
CNCHUNKS = T_EDGES // CCB


def _node_mlp_kernel(x_ref, w1a_ref, w1b_ref, b1_ref, xa_ref, xb_ref):
    x = x_ref[...]
    xa_ref[...] = jnp.dot(x, w1a_ref[...], preferred_element_type=jnp.float32) + b1_ref[...]
    xb_ref[...] = jnp.dot(x, w1b_ref[...], preferred_element_type=jnp.float32)


def _edge_mlp_kernel(ea_ref, w1c_ref, out_ref):
    out_ref[...] = jnp.dot(ea_ref[...], w1c_ref[...], preferred_element_type=jnp.float32)


def _sc_count_kernel(dst_hbm, cnt_hbm, idxc0, idxc1, ones_b, cnt_sh,
                     sem_i0, sem_i1, sem_s0, sem_s1):
    cx = lax.axis_index("c")
    sx = lax.axis_index("s")
    tile = cx * NSUB + sx
    ebase = tile * T_EDGES
    rbase = sx * ROWS_PER_TILE
    idxc = (idxc0, idxc1)
    sem_i = (sem_i0, sem_i1)
    sem_s = (sem_s0, sem_s1)

    def i_desc(c, p):
        return pltpu.make_async_copy(
            dst_hbm.at[pl.ds(ebase + c * CCB, CCB)], idxc[p], sem_i[p])

    def s_desc(p):
        return pltpu.make_async_copy(ones_b, cnt_sh.at[idxc[p]], sem_s[p])

    @pl.loop(0, CCB)
    def _(r):
        ones_b.at[pl.ds(r, 1), pl.ds(0, CW)][...] = jnp.zeros((1, CW), jnp.float32)
    for k in range(ROWS_PER_TILE // CCB):
        pltpu.sync_copy(ones_b, cnt_sh.at[pl.ds(rbase + k * CCB, CCB), :])

    @pl.loop(0, CCB)
    def _(r):
        ones_b.at[pl.ds(r, 1), pl.ds(0, CW)][...] = jnp.full((1, CW), 1.0, jnp.float32)
    plsc.subcore_barrier()

    i_desc(0, 0).start()

    @pl.loop(0, CNCHUNKS // 2)
    def _(t):
        for u in range(2):
            c = 2 * t + u
            p = u
            p1 = 1 - u
            i_desc(c, p).wait()
            s_desc(p).start(add=True)

            def sw(p1=p1):
                s_desc(p1).wait()

            if u == 0:
                pl.when(t > 0)(sw)
            else:
                sw()

            def si(c=c, p1=p1):
                i_desc(c + 1, p1).start()

            if u == 1:
                pl.when(t < CNCHUNKS // 2 - 1)(si)
            else:
                si()

    s_desc(1).wait()
    plsc.subcore_barrier()
    pltpu.sync_copy(cnt_sh.at[pl.ds(rbase, ROWS_PER_TILE), :],
                    cnt_hbm.at[pl.ds(cx * N_ACC + rbase, ROWS_PER_TILE), :])


def _sc_edge_kernel(dst_hbm, src_hbm, ea_hbm, xa_hbm, xb_hbm,
                    hacc_hbm,
                    idxd0, idxd1, idxd2, idxd3,
                    idxs0, idxs1, idxs2, idxs3,
                    bufa0, bufa1, bufe0, bufe1,
                    acc_sh,
                    sem_i0, sem_i1, sem_i2, sem_i3,
                    sem_e0, sem_e1, sem_a0, sem_a1,
                    sem_b0, sem_b1, sem_s0, sem_s1):
    cx = lax.axis_index("c")
    sx = lax.axis_index("s")
    tile = cx * NSUB + sx
    ebase = tile * T_EDGES

    idxd = (idxd0, idxd1, idxd2, idxd3)
    idxs = (idxs0, idxs1, idxs2, idxs3)
    bufa = (bufa0, bufa1)
    bufe = (bufe0, bufe1)
    sem_i = (sem_i0, sem_i1, sem_i2, sem_i3)
    sem_e = (sem_e0, sem_e1)
    sem_a = (sem_a0, sem_a1)
    sem_b = (sem_b0, sem_b1)
    sem_s = (sem_s0, sem_s1)

    def i_descs(c, q):
        return (
            pltpu.make_async_copy(dst_hbm.at[pl.ds(ebase + c * CB, CB)], idxd[q], sem_i[q]),
            pltpu.make_async_copy(src_hbm.at[pl.ds(ebase + c * CB, CB)], idxs[q], sem_i[q]),
        )

    def e_desc(c, p):
        return pltpu.make_async_copy(
            ea_hbm.at[pl.ds(ebase + c * CB, CB), :], bufe[p], sem_e[p])

    def a_desc(q, p):
        return pltpu.make_async_copy(xa_hbm.at[idxd[q]], bufa[p], sem_a[p])

    def b_desc(q, p):
        return pltpu.make_async_copy(xb_hbm.at[idxs[q]], bufe[p], sem_b[p])

    def s_desc(q, p):
        return pltpu.make_async_copy(bufa[p], acc_sh.at[idxd[q]], sem_s[p])

    @pl.loop(0, CB)
    def _(r):
        for cc in range(NODE_DIM // 16):
            bufa0.at[pl.ds(r, 1), pl.ds(cc * 16, 16)][...] = jnp.zeros((1, 16), jnp.float32)
    rbase = sx * ROWS_PER_TILE
    for k in range(ROWS_PER_TILE // CB):
        pltpu.sync_copy(bufa0, acc_sh.at[pl.ds(rbase + k * CB, CB), :])
    plsc.subcore_barrier()

    def compute(p):
        @pl.loop(0, CB)
        def _(r):
            for cc in range(NODE_DIM // 16):
                slc = (pl.ds(r, 1), pl.ds(cc * 16, 16))
                bufa[p].at[slc][...] = jnp.maximum(
                    bufa[p].at[slc][...] + bufe[p].at[slc][...], 0.0)

    for d in i_descs(0, 0):
        d.start()
    for d in i_descs(1, 1):
        d.start()
    for d in i_descs(2, 2):
        d.start()
    e_desc(0, 0).start()
    e_desc(1, 1).start()
    for d in i_descs(0, 0):
        d.wait()
    a_desc(0, 0).start()
    e_desc(0, 0).wait()
    b_desc(0, 0).start(add=True)

    @pl.loop(0, NT)
    def _(t):
        for u in range(4):
            c = 4 * t + u
            p = u % 2
            p1 = (u + 1) % 2
            q1 = (u + 1) % 4
            qm1 = (u + 3) % 4

            def s1(c=c, p1=p1, q1=q1):
                for d in i_descs(c + 1, q1):
                    d.wait()
                e_desc(c + 1, p1).wait()
                b_desc(q1, p1).start(add=True)

            if u == 3:
                pl.when(t < NT - 1)(s1)
            else:
                s1()

            def s2w(p1=p1, qm1=qm1):
                s_desc(qm1, p1).wait()

            if u == 0:
                pl.when(t > 0)(s2w)
            else:
                s2w()

            def s2a(q1=q1, p1=p1):
                a_desc(q1, p1).start()

            if u == 3:
                pl.when(t < NT - 1)(s2a)
            else:
                s2a()

            def s3(c=c, u=u):
                for d in i_descs(c + 3, (u + 3) % 4):
                    d.start()

            if u == 0:
                s3()
            else:
                pl.when(t < NT - 1)(s3)

            a_desc(u, p).wait()
            b_desc(u, p).wait()
            compute(p)
            s_desc(u, p).start(add=True)

            def s7(c=c, p=p):
                e_desc(c + 2, p).start()

            if u <= 1:
                s7()
            else:
                pl.when(t < NT - 1)(s7)

    s_desc(3, 1).wait()
    plsc.subcore_barrier()
    pltpu.sync_copy(acc_sh.at[pl.ds(rbase, ROWS_PER_TILE), :],
                    hacc_hbm.at[pl.ds(cx * N_ACC + rbase, ROWS_PER_TILE), :])


def _combine_kernel(hacc_ref, cnt_ref, w2_ref, b2_ref, out_ref):
    h = hacc_ref[:N_NODES, :] + hacc_ref[N_ACC:N_ACC + N_NODES, :]
    cnt = cnt_ref[:N_NODES, 0:1] + cnt_ref[N_ACC:N_ACC + N_NODES, 0:1]
    out_ref[...] = (
        jnp.dot(h, w2_ref[...], preferred_element_type=jnp.float32) + cnt * b2_ref[...]
    )


def kernel(x, edge_index, edge_attr, W1, b1, W2, b2):
    dst = edge_index[1].astype(jnp.int32)
    src = edge_index[0].astype(jnp.int32)
    W1a = W1[:NODE_DIM]
    W1b = W1[NODE_DIM:2 * NODE_DIM]
    W1c = W1[2 * NODE_DIM:]
    b1r = b1.reshape(1, HIDDEN)
    b2r = b2.reshape(1, NODE_DIM)

    per_tile = N_EDGES // NTILES
    pad_n = T_EDGES - per_tile
    dst_f = jnp.pad(dst.reshape(NTILES, per_tile), ((0, 0), (0, pad_n)),
                    constant_values=N_NODES).reshape(E_PAD)
    src_f = jnp.pad(src.reshape(NTILES, per_tile), ((0, 0), (0, pad_n)),
                    constant_values=N_NODES).reshape(E_PAD)
    ea_in = jnp.pad(edge_attr.reshape(NTILES, per_tile, EDGE_DIM),
                    ((0, 0), (0, pad_n), (0, 0))).reshape(E_PAD, EDGE_DIM)

    xa, xb = pl.pallas_call(
        _node_mlp_kernel,
        out_shape=[
            jax.ShapeDtypeStruct((N_NODES, HIDDEN), jnp.float32),
            jax.ShapeDtypeStruct((N_NODES, HIDDEN), jnp.float32),
        ],
    )(x, W1a, W1b, b1r)
    xa = jnp.pad(xa, ((0, N_TAB - N_NODES), (0, 0)))
    xb = jnp.pad(xb, ((0, N_TAB - N_NODES), (0, 0)))

    ea = pl.pallas_call(
        _edge_mlp_kernel,
        grid=(E_PAD // 20480,),
        in_specs=[
            pl.BlockSpec((20480, EDGE_DIM), lambda k: (k, 0)),
            pl.BlockSpec((EDGE_DIM, HIDDEN), lambda k: (0, 0)),
        ],
        out_specs=pl.BlockSpec((20480, HIDDEN), lambda k: (k, 0)),
        out_shape=jax.ShapeDtypeStruct((E_PAD, HIDDEN), jnp.float32),
    )(ea_in, W1c)

    mesh = plsc.VectorSubcoreMesh(core_axis_name="c", subcore_axis_name="s")
    sc_count = pl.kernel(
        _sc_count_kernel,
        out_type=jax.ShapeDtypeStruct((NCORES * N_ACC, CW), jnp.float32),
        mesh=mesh,
        scratch_types=[
            pltpu.VMEM((CCB,), jnp.int32),
            pltpu.VMEM((CCB,), jnp.int32),
            pltpu.VMEM((CCB, CW), jnp.float32),
            pltpu.VMEM_SHARED((N_ACC, CW), jnp.float32),
            pltpu.SemaphoreType.DMA,
            pltpu.SemaphoreType.DMA,
            pltpu.SemaphoreType.DMA,
            pltpu.SemaphoreType.DMA,
        ],
    )
    cnt = sc_count(dst_f)

    sc_edge = pl.kernel(
        _sc_edge_kernel,
        out_type=jax.ShapeDtypeStruct((NCORES * N_ACC, HIDDEN), jnp.float32),
        mesh=mesh,
        scratch_types=(
            [pltpu.VMEM((CB,), jnp.int32)] * 8
            + [pltpu.VMEM((CB, HIDDEN), jnp.float32)] * 4
            + [pltpu.VMEM_SHARED((N_ACC, HIDDEN), jnp.float32)]
            + [pltpu.SemaphoreType.DMA] * 12
        ),
    )
    hacc = sc_edge(dst_f, src_f, ea, xa, xb)

    out = pl.pallas_call(
        _combine_kernel,
        out_shape=jax.ShapeDtypeStruct((N_NODES, NODE_DIM), jnp.float32),
    )(hacc, cnt, W2, b2r)
    return out
